# Initial kernel scaffold; baseline (speedup 1.0000x reference)
import jax
import jax.numpy as jnp
from jax.experimental import pallas as pl


def kernel(x, edge_index_0, edge_index_1, edge_index_2, W0, b0, W1, b1, W2, b2, alphas, lin_W, lin_b):
    raise NotImplementedError("write your pallas kernel here")


# speedup vs baseline: 11.7146x; 11.7146x over previous
import functools

import jax
import jax.numpy as jnp
from jax import lax
from jax.experimental import pallas as pl
from jax.experimental.pallas import tpu as pltpu
from jax.experimental.pallas import tpu_sc as plsc

N = 10000
NP = 10240
D = 128
E = 320000
NC = 2
NS = 16
NW = NC * NS
EPT = E // NW
CK = 128
CH = NP // NW * 4
CHUNKS = 80
HCH = CHUNKS // 2
EPTP = CHUNKS * CK
RPT = NP // NS
RB = 1024
GRID = NP // RB


def _deg_body(s0, d0, s1, d1, s2, d2, out_hbm, idx_v, hist_v):
    c = lax.axis_index("c")
    s = lax.axis_index("s")
    wid = s * NC + c
    ones = jnp.full((16,), 1.0, dtype=jnp.float32)
    zeros = jnp.zeros((16,), dtype=jnp.float32)
    for row, ref in enumerate((s0, d0, s1, d1, s2, d2)):
        pltpu.sync_copy(ref.at[wid], idx_v)

        def zero_body(j, carry):
            hist_v[pl.ds(j * 16, 16)] = zeros
            return carry

        lax.fori_loop(0, NP // 16, zero_body, 0)

        def add_body(j, carry):
            v = idx_v[pl.ds(j * 16, 16)]
            plsc.addupdate_scatter(hist_v, [v], ones)
            return carry

        lax.fori_loop(0, EPT // 16, add_body, 0)
        pltpu.sync_copy(hist_v, out_hbm.at[row, wid])


def _make_deg_kernel():
    mesh = plsc.VectorSubcoreMesh(
        core_axis_name="c", subcore_axis_name="s",
        num_cores=NC, num_subcores=NS,
    )
    return pl.kernel(
        _deg_body,
        out_type=jax.ShapeDtypeStruct((6, NW, NP), jnp.float32),
        mesh=mesh,
        scratch_types=[
            pltpu.VMEM((EPT,), jnp.int32),
            pltpu.VMEM((NP,), jnp.float32),
        ],
        compiler_params=pltpu.CompilerParams(needs_layout_passes=False),
    )


def _norm_mm_body(degp_ref, x_ref, w_ref, norm_ref, h0_ref, h1_ref, h2_ref):
    deg = jnp.sum(degp_ref[...], axis=1)
    safe = jnp.where(deg > 0.0, deg, 1.0)
    norm = jnp.where(deg > 0.0, lax.rsqrt(safe), 0.0)
    norm_ref[...] = jnp.concatenate(
        [norm, jnp.zeros((2, RB), jnp.float32)], axis=0
    )
    xb = x_ref[...]
    for g, h_ref in enumerate((h0_ref, h1_ref, h2_ref)):
        xs = xb * norm[2 * g][:, None]
        h_ref[...] = jnp.dot(
            xs, w_ref[g], preferred_element_type=jnp.float32
        )


def _norm_mm_call(degp, x_pad, w_all):
    return pl.pallas_call(
        _norm_mm_body,
        grid=(GRID,),
        in_specs=[
            pl.BlockSpec((6, NW, RB), lambda i: (0, 0, i)),
            pl.BlockSpec((RB, D), lambda i: (i, 0)),
            pl.BlockSpec((3, D, D), lambda i: (0, 0, 0)),
        ],
        out_specs=[
            pl.BlockSpec((8, RB), lambda i: (0, i)),
            pl.BlockSpec((RB, D), lambda i: (i, 0)),
            pl.BlockSpec((RB, D), lambda i: (i, 0)),
            pl.BlockSpec((RB, D), lambda i: (i, 0)),
        ],
        out_shape=[
            jax.ShapeDtypeStruct((8, NP), jnp.float32),
            jax.ShapeDtypeStruct((NP, D), jnp.float32),
            jax.ShapeDtypeStruct((NP, D), jnp.float32),
            jax.ShapeDtypeStruct((NP, D), jnp.float32),
        ],
    )(degp, x_pad, w_all)


def _scatter_body(h0, h1, h2, s0, d0, s1, d1, s2, d2, zeros_hbm,
                  o0, o1, o2, src_v, dst_v, rows_a, rows_b, agg_sh,
                  sem_a, sem_b):
    c = lax.axis_index("c")
    s = lax.axis_index("s")
    wid = s * NC + c
    for g, (h_ref, s_ref, d_ref, o_ref) in enumerate(
        ((h0, s0, d0, o0), (h1, s1, d1, o1), (h2, s2, d2, o2))
    ):
        pltpu.sync_copy(zeros_hbm, agg_sh.at[pl.ds(s * RPT, RPT)])
        plsc.subcore_barrier()

        for half in range(2):
            pltpu.sync_copy(s_ref.at[wid, pl.ds(half * HCH, HCH)], src_v)
            pltpu.sync_copy(d_ref.at[wid, pl.ds(half * HCH, HCH)], dst_v)

            pltpu.make_async_copy(h_ref.at[src_v.at[0]], rows_a, sem_a).start()
            pltpu.make_async_copy(h_ref.at[src_v.at[1]], rows_b, sem_b).start()

            def chunk_body(j, carry):
                ch = j * 2
                pltpu.make_async_copy(
                    h_ref.at[src_v.at[ch]], rows_a, sem_a
                ).wait()

                @pl.when(ch + 2 < HCH)
                def _():
                    pltpu.make_async_copy(
                        h_ref.at[src_v.at[ch + 2]], rows_a, sem_a
                    ).start()

                pltpu.sync_copy(rows_a, agg_sh.at[dst_v.at[ch]], add=True)

                pltpu.make_async_copy(
                    h_ref.at[src_v.at[ch + 1]], rows_b, sem_b
                ).wait()

                @pl.when(ch + 3 < HCH)
                def _():
                    pltpu.make_async_copy(
                        h_ref.at[src_v.at[ch + 3]], rows_b, sem_b
                    ).start()

                pltpu.sync_copy(rows_b, agg_sh.at[dst_v.at[ch + 1]], add=True)
                return carry

            lax.fori_loop(0, HCH // 2, chunk_body, 0)
        plsc.subcore_barrier()
        pltpu.sync_copy(
            agg_sh.at[pl.ds(s * RPT, RPT)],
            o_ref.at[c, pl.ds(s * RPT, RPT)],
        )


def _make_scatter_kernel():
    mesh = plsc.VectorSubcoreMesh(
        core_axis_name="c", subcore_axis_name="s",
        num_cores=NC, num_subcores=NS,
    )
    agg_shape = jax.ShapeDtypeStruct((NC, NP, D), jnp.float32)
    return pl.kernel(
        _scatter_body,
        out_type=[agg_shape, agg_shape, agg_shape],
        mesh=mesh,
        scratch_types=[
            pltpu.VMEM((HCH, CK), jnp.int32),
            pltpu.VMEM((HCH, CK), jnp.int32),
            pltpu.VMEM((CK, D), jnp.float32),
            pltpu.VMEM((CK, D), jnp.float32),
            pltpu.VMEM_SHARED((NP, D), jnp.float32),
            pltpu.SemaphoreType.DMA,
            pltpu.SemaphoreType.DMA,
        ],
    )


def _finish_body(a0_ref, a1_ref, a2_ref, norm_ref, b_ref, alpha_ref,
                 lw_ref, lb_ref, out_ref):
    e0 = jnp.exp(alpha_ref[0, 0])
    e1 = jnp.exp(alpha_ref[0, 1])
    e2 = jnp.exp(alpha_ref[0, 2])
    inv = 1.0 / (e0 + e1 + e2)
    aw = (e0 * inv, e1 * inv, e2 * inv)
    acc = jnp.broadcast_to(lb_ref[0], (RB, D))
    for g, agg_ref in enumerate((a0_ref, a1_ref, a2_ref)):
        agg = agg_ref[0] + agg_ref[1]
        nd = norm_ref[2 * g + 1][:, None]
        t = jnp.tanh(aw[g] * (agg * nd + b_ref[g][None, :]))
        acc = acc + jnp.dot(
            t, lw_ref[g], preferred_element_type=jnp.float32
        )
    out_ref[...] = acc


def _finish_call(agg0, agg1, agg2, norms, b_all, alphas2d, lw3, lin_b2d):
    return pl.pallas_call(
        _finish_body,
        grid=(GRID,),
        in_specs=[
            pl.BlockSpec((NC, RB, D), lambda i: (0, i, 0)),
            pl.BlockSpec((NC, RB, D), lambda i: (0, i, 0)),
            pl.BlockSpec((NC, RB, D), lambda i: (0, i, 0)),
            pl.BlockSpec((8, RB), lambda i: (0, i)),
            pl.BlockSpec((3, D), lambda i: (0, 0)),
            pl.BlockSpec(memory_space=pltpu.SMEM),
            pl.BlockSpec((3, D, D), lambda i: (0, 0, 0)),
            pl.BlockSpec((1, D), lambda i: (0, 0)),
        ],
        out_specs=pl.BlockSpec((RB, D), lambda i: (i, 0)),
        out_shape=jax.ShapeDtypeStruct((NP, D), jnp.float32),
    )(agg0, agg1, agg2, norms, b_all, alphas2d, lw3, lin_b2d)


def _prep_edges(ei):
    src = ei[0].reshape(NW, EPT)
    dst = ei[1].reshape(NW, EPT)
    pad_s = jnp.broadcast_to(
        (jnp.arange(NW, dtype=jnp.int32) % 16)[:, None], (NW, EPTP - EPT)
    )
    pad_d = pad_s + N
    src_p = jnp.concatenate([src, pad_s], axis=1).reshape(NW, CHUNKS, CK)
    dst_p = jnp.concatenate([dst, pad_d], axis=1).reshape(NW, CHUNKS, CK)
    return src, dst, src_p, dst_p


@jax.jit
def kernel(x, edge_index_0, edge_index_1, edge_index_2,
           W0, b0, W1, b1, W2, b2, alphas, lin_W, lin_b):
    s0, d0, s0p, d0p = _prep_edges(edge_index_0)
    s1, d1, s1p, d1p = _prep_edges(edge_index_1)
    s2, d2, s2p, d2p = _prep_edges(edge_index_2)

    degp = _make_deg_kernel()(s0, d0, s1, d1, s2, d2)

    x_pad = jnp.pad(x, ((0, NP - N), (0, 0)))
    w_all = jnp.stack([W0, W1, W2])
    norms, h0, h1, h2 = _norm_mm_call(degp, x_pad, w_all)

    zeros_hbm = jnp.zeros((RPT, D), jnp.float32)
    agg0, agg1, agg2 = _make_scatter_kernel()(
        h0, h1, h2, s0p, d0p, s1p, d1p, s2p, d2p, zeros_hbm
    )

    b_all = jnp.stack([b0, b1, b2])
    lw3 = lin_W.reshape(3, D, D)
    out = _finish_call(
        agg0, agg1, agg2, norms, b_all,
        alphas.reshape(1, 3), lw3, lin_b.reshape(1, D),
    )
    return out[:N]
